# baseline (speedup 1.0000x reference)
import functools
import math

import jax
import jax.numpy as jnp
from jax import lax
from jax.experimental import pallas as pl
from jax.experimental.pallas import tpu as pltpu
from jax.experimental.pallas import tpu_sc as plsc

VOCAB_SIZE = 1000000
D = 64
DP = 128
SCALE = math.sqrt(D)
NC, NS = 2, 16
NW = NC * NS
ROWS = 4096
SEQ = 200
R_PER_W = ROWS // NW
IDX_PER_W = R_PER_W * SEQ

BLK = 8000


def _prescale_body(tab_ref, out_ref):
    out_ref[:, :D] = tab_ref[...] * SCALE


_prescale = pl.pallas_call(
    _prescale_body,
    grid=(VOCAB_SIZE // BLK,),
    in_specs=[pl.BlockSpec((BLK, D), lambda i: (i, 0))],
    out_specs=pl.BlockSpec((BLK, DP), lambda i: (i, 0)),
    out_shape=jax.ShapeDtypeStruct((VOCAB_SIZE, DP), jnp.float32),
)


def _make_sc_kernel():
    mesh = plsc.VectorSubcoreMesh(core_axis_name="c", subcore_axis_name="s")

    @functools.partial(
        pl.kernel,
        out_type=jax.ShapeDtypeStruct((ROWS, SEQ, D), jnp.float32),
        mesh=mesh,
        compiler_params=pltpu.CompilerParams(use_tc_tiling_on_sc=False),
        scratch_types=[
            pltpu.VMEM((IDX_PER_W,), jnp.int32),
            pltpu.VMEM((SEQ, DP), jnp.float32),
            pltpu.VMEM((SEQ, DP), jnp.float32),
            pltpu.SemaphoreType.DMA,
            pltpu.SemaphoreType.DMA,
        ],
    )
    def emb(x_hbm, tab_hbm, out_hbm, idx_v, buf0, buf1, sem0, sem1):
        wid = lax.axis_index("s") * NC + lax.axis_index("c")
        r0 = wid * R_PER_W
        pltpu.sync_copy(x_hbm.at[pl.ds(wid * IDX_PER_W, IDX_PER_W)], idx_v)
        bufs = (buf0, buf1)
        sems = (sem0, sem1)

        def issue(r, b):
            for off, n in ((0, 128), (128, 72)):
                pltpu.async_copy(
                    tab_hbm.at[idx_v.at[pl.ds(r * SEQ + off, n)]],
                    bufs[b].at[pl.ds(off, n)],
                    sems[b])

        def wait_gathers(r, b):
            for off, n in ((0, 128), (128, 72)):
                pltpu.make_async_copy(
                    tab_hbm.at[idx_v.at[pl.ds(r * SEQ + off, n)]],
                    bufs[b].at[pl.ds(off, n)],
                    sems[b]).wait()

        for b in range(2):
            issue(b, b)

        @pl.loop(0, R_PER_W, step=2)
        def _row(r):
            for b in range(2):
                rr = r + b
                wait_gathers(rr, b)
                pltpu.sync_copy(bufs[b].at[:, pl.ds(0, D)],
                                out_hbm.at[r0 + rr])

                @pl.when(rr + 2 < R_PER_W)
                def _next():
                    issue(rr + 2, b)

    return emb


_emb = _make_sc_kernel()


def kernel(x, table):
    tab2 = _prescale(table)
    xf = x.astype(jnp.int32).reshape(-1)
    return _emb(xf, tab2)
